# baseline (speedup 1.0000x reference)
import functools

import jax
import jax.numpy as jnp
from jax import lax
from jax.experimental import pallas as pl
from jax.experimental.pallas import tpu as pltpu
from jax.experimental.pallas import tpu_sc as plsc

_D = 64
_DPAD = 128
_BATCH = 4096
_SEQ = 200

_info = plsc.get_sparse_core_info()
_NC, _NS, _L = _info.num_cores, _info.num_subcores, _info.num_lanes
_NW = _NC * _NS
_BBLK = _BATCH // _NW
_NIB = 4
_NRB = 4
_DG = 3


def _embed_body(idx_hbm, table_hbm, out_hbm, idx_v, rows_v, comp_v,
                isem, gsem, osem):
    wid = lax.axis_index("s") * _NC + lax.axis_index("c")
    b0 = wid * _BBLK

    def idx_copy(s, slot):
        return pltpu.make_async_copy(idx_hbm.at[s, pl.ds(b0, _BBLK)],
                                     idx_v.at[slot], isem.at[slot])

    def gather(islot, rslot):
        return pltpu.make_async_copy(table_hbm.at[idx_v.at[islot]],
                                     rows_v.at[rslot], gsem.at[rslot])

    def out_copy(s, slot):
        return pltpu.make_async_copy(
            comp_v.at[slot], out_hbm.at[s, :, pl.ds(b0, _BBLK)],
            osem.at[slot])

    bases = [lax.iota(jnp.int32, _L) + (_L * k) for k in range(_BBLK // _L)]

    def transpose_chunk(slot):
        rows = rows_v.at[slot]

        @plsc.parallel_loop(0, _D, unroll=8, carry=jnp.zeros((_L,), jnp.int32))
        def _(d, col):
            for k in range(_BBLK // _L):
                comp_v[slot, d, pl.ds(_L * k, _L)] = plsc.load_gather(
                    rows, [bases[k], col])
            return col + 1

    for s in range(_NIB):
        idx_copy(s, s).start()
    for s in range(_DG):
        idx_copy(s, s).wait()
        gather(s, s).start()

    def step(i, carry):
        s0 = i * _NIB
        for j in range(_NIB):
            s = s0 + j
            r = j % _NRB
            gather(j, r).wait()

            @pl.when(s >= _NRB)
            def _():
                out_copy(0, r).wait()

            transpose_chunk(r)
            out_copy(s, r).start()
            si = s + _NIB

            @pl.when(si < _SEQ)
            def _():
                idx_copy(si, j).start()

            sg = s + _DG
            islot = (j + _DG) % _NIB
            rslot = (j + _DG) % _NRB

            @pl.when(sg < _SEQ)
            def _():
                idx_copy(0, islot).wait()
                gather(islot, rslot).start()

        return carry

    lax.fori_loop(0, _SEQ // _NIB, step, 0)

    for j in range(_NRB):
        out_copy(0, j).wait()


_mesh = plsc.VectorSubcoreMesh(core_axis_name="c", subcore_axis_name="s")

_embed = functools.partial(
    pl.kernel,
    mesh=_mesh,
    out_type=jax.ShapeDtypeStruct((_SEQ, _D, _BATCH), jnp.float32),
    scratch_types=[
        pltpu.VMEM((_NIB, _BBLK), jnp.int32),
        pltpu.VMEM((_NRB, _BBLK, _DPAD), jnp.float32),
        pltpu.VMEM((_NRB, _D, _BBLK), jnp.float32),
        pltpu.SemaphoreType.DMA((_NIB,)),
        pltpu.SemaphoreType.DMA((_NRB,)),
        pltpu.SemaphoreType.DMA((_NRB,)),
    ],
    compiler_params=pltpu.CompilerParams(use_tc_tiling_on_sc=True,
                                         needs_layout_passes=False,
                                         disable_bounds_checks=True),
)(_embed_body)


@jax.jit
def kernel(actions, table):
    idx_t = actions.T.astype(jnp.int32)
    tab = jnp.pad(table, ((0, 0), (0, _DPAD - _D)))
    out_t = _embed(idx_t, tab)
    return jnp.transpose(out_t, (2, 0, 1))


# speedup vs baseline: 1.0163x; 1.0163x over previous
import functools

import jax
import jax.numpy as jnp
from jax import lax
from jax.experimental import pallas as pl
from jax.experimental.pallas import tpu as pltpu
from jax.experimental.pallas import tpu_sc as plsc

_D = 64
_B_TOTAL = 4096 * 200

_info = plsc.get_sparse_core_info()
_NC, _NS = _info.num_cores, _info.num_subcores
_NW = _NC * _NS
_B_PER_W = _B_TOTAL // _NW
_CHUNK = 256
_NCHUNKS = _B_PER_W // _CHUNK
_NBUF = 4
_DI = 4
_DG = 2


def _embed_body(idx_hbm, table_hbm, out_hbm, idx_v, rows_v, isem, gsem, osem):
    wid = lax.axis_index("s") * _NC + lax.axis_index("c")
    base = wid * _B_PER_W

    def idx_copy(g, slot):
        return pltpu.make_async_copy(idx_hbm.at[wid, g], idx_v.at[slot],
                                     isem.at[slot])

    def gather(slot):
        return pltpu.make_async_copy(table_hbm.at[idx_v.at[slot]],
                                     rows_v.at[slot], gsem.at[slot])

    def out_copy(g, slot):
        return pltpu.make_async_copy(
            rows_v.at[slot], out_hbm.at[pl.ds(base + g * _CHUNK, _CHUNK)],
            osem.at[slot])

    for g in range(_DI):
        idx_copy(g, g % _NBUF).start()
    for g in range(_DG):
        idx_copy(g, g % _NBUF).wait()
        gather(g % _NBUF).start()

    def step(i, carry):
        g0 = i * _NBUF
        for j in range(_NBUF):
            g = g0 + j
            gather(j).wait()
            out_copy(g, j).start()
            gi = g + _DI

            @pl.when(gi < _NCHUNKS)
            def _():
                idx_copy(gi, j).start()

            gg = g + _DG
            gslot = (j + _DG) % _NBUF

            @pl.when(gg < _NCHUNKS)
            def _():
                @pl.when(gg >= _NBUF)
                def _():
                    out_copy(0, gslot).wait()
                idx_copy(0, gslot).wait()
                gather(gslot).start()

        return carry

    lax.fori_loop(0, _NCHUNKS // _NBUF, step, 0)

    for j in range(_NBUF):
        out_copy(0, j).wait()


_mesh = plsc.VectorSubcoreMesh(core_axis_name="c", subcore_axis_name="s")

_embed = functools.partial(
    pl.kernel,
    mesh=_mesh,
    out_type=jax.ShapeDtypeStruct((_B_TOTAL, _D), jnp.float32),
    scratch_types=[
        pltpu.VMEM((_NBUF, _CHUNK), jnp.int32),
        pltpu.VMEM((_NBUF, _CHUNK, _D), jnp.float32),
        pltpu.SemaphoreType.DMA((_NBUF,)),
        pltpu.SemaphoreType.DMA((_NBUF,)),
        pltpu.SemaphoreType.DMA((_NBUF,)),
    ],
    compiler_params=pltpu.CompilerParams(use_tc_tiling_on_sc=False),
)(_embed_body)


@jax.jit
def kernel(actions, table):
    idx = actions.reshape(_NW, _NCHUNKS, _CHUNK).astype(jnp.int32)
    out = _embed(idx, table)
    return out.reshape(actions.shape[0], actions.shape[1], _D)
